# baseline (speedup 1.0000x reference)
import functools

import jax
import jax.numpy as jnp
from jax import lax
from jax.experimental import pallas as pl
from jax.experimental.pallas import tpu as pltpu
from jax.experimental.pallas import tpu_sc as plsc

L = 4096
SC_ROWS = 2560
TC_ROWS = L - SC_ROWS
NUM_WORKERS = 32
ROWS_PER_WORKER = SC_ROWS // NUM_WORKERS
ROWS_PER_CHUNK = 1
NUM_CHUNKS = ROWS_PER_WORKER // ROWS_PER_CHUNK
VECS_PER_ROW = L // 16
NBUF = 8
ROUNDS = NUM_CHUNKS // NBUF
TC_BLK = 256


def _sc_lookup(table16, idx):
    mesh = plsc.VectorSubcoreMesh(core_axis_name="c", subcore_axis_name="s")

    scratch = [pltpu.VMEM((16,), jnp.float32)]
    scratch += [pltpu.VMEM((ROWS_PER_CHUNK, L), jnp.int32)] * NBUF
    scratch += [pltpu.VMEM((ROWS_PER_CHUNK, L), jnp.float32)] * NBUF
    scratch += [pltpu.SemaphoreType.DMA] * (2 * NBUF)

    @functools.partial(
        pl.kernel,
        mesh=mesh,
        out_type=jax.ShapeDtypeStruct((SC_ROWS, L), jnp.float32),
        compiler_params=pltpu.CompilerParams(needs_layout_passes=False),
        scratch_types=scratch,
    )
    def k(table_hbm, idx_hbm, out_hbm, tab_v, *bufs):
        idx_b = bufs[:NBUF]
        out_b = bufs[NBUF:2 * NBUF]
        sin = bufs[2 * NBUF:3 * NBUF]
        sout = bufs[3 * NBUF:4 * NBUF]
        wid = lax.axis_index("s") * 2 + lax.axis_index("c")
        row_base = wid * ROWS_PER_WORKER
        pltpu.sync_copy(table_hbm, tab_v)
        tab_vec = tab_v[...]

        for b in range(NBUF):
            pltpu.async_copy(
                idx_hbm.at[pl.ds(row_base + b * ROWS_PER_CHUNK,
                                 ROWS_PER_CHUNK)],
                idx_b[b], sin[b])

        def round_body(r, carry):
            for b in range(NBUF):
                ci = r * NBUF + b
                r0 = row_base + ci * ROWS_PER_CHUNK
                pltpu.make_async_copy(
                    idx_hbm.at[pl.ds(r0, ROWS_PER_CHUNK)],
                    idx_b[b], sin[b]).wait()

                @pl.when(r > 0)
                def _wait_out():
                    pltpu.make_async_copy(
                        out_b[b], out_hbm.at[pl.ds(r0, ROWS_PER_CHUNK)],
                        sout[b]).wait()

                for row in range(ROWS_PER_CHUNK):
                    @plsc.parallel_loop(0, VECS_PER_ROW, 1, unroll=8)
                    def _vec(vi):
                        s = pl.ds(vi * 16, 16)
                        iv = idx_b[b][row, s]
                        out_b[b][row, s] = tab_vec.at[iv].get(
                            mode="promise_in_bounds")

                pltpu.async_copy(
                    out_b[b], out_hbm.at[pl.ds(r0, ROWS_PER_CHUNK)], sout[b])

                @pl.when(ci + NBUF < NUM_CHUNKS)
                def _prefetch():
                    r2 = row_base + (ci + NBUF) * ROWS_PER_CHUNK
                    pltpu.async_copy(
                        idx_hbm.at[pl.ds(r2, ROWS_PER_CHUNK)],
                        idx_b[b], sin[b])

            return carry

        lax.fori_loop(0, ROUNDS, round_body, 0)

        for b in range(NBUF):
            pltpu.make_async_copy(
                out_b[b], out_hbm.at[pl.ds(row_base, ROWS_PER_CHUNK)],
                sout[b]).wait()

    return k(table16, idx)


def _tc_body(tab_ref, idx_ref, out_ref):
    idx = idx_ref[...]
    t = [tab_ref[0, k] for k in range(16)]
    b0 = (idx & 1) == 1
    l0 = [jnp.where(b0, t[2 * k + 1], t[2 * k]) for k in range(8)]
    b1 = (idx & 2) == 2
    l1 = [jnp.where(b1, l0[2 * k + 1], l0[2 * k]) for k in range(4)]
    b2 = (idx & 4) == 4
    l2 = [jnp.where(b2, l1[2 * k + 1], l1[2 * k]) for k in range(2)]
    b3 = (idx & 8) == 8
    out_ref[...] = jnp.where(b3, l2[1], l2[0])


def _tc_lookup(table16, idx_full):
    return pl.pallas_call(
        _tc_body,
        grid=(TC_ROWS // TC_BLK,),
        in_specs=[
            pl.BlockSpec((1, 16), lambda i: (0, 0)),
            pl.BlockSpec((TC_BLK, L), lambda i: (i + SC_ROWS // TC_BLK, 0)),
        ],
        out_specs=pl.BlockSpec((TC_BLK, L), lambda i: (i, 0)),
        out_shape=jax.ShapeDtypeStruct((TC_ROWS, L), jnp.float32),
    )(table16.reshape(1, 16), idx_full)


def kernel(selected_ids, crf_transitions_model):
    idx = selected_ids.astype(jnp.int32)
    flat = crf_transitions_model.reshape(-1)
    table16 = jnp.concatenate([flat, jnp.zeros((1,), jnp.float32)])
    sc_out = _sc_lookup(table16, idx)
    tc_out = _tc_lookup(table16, idx)
    return jnp.concatenate([sc_out, tc_out], axis=0)


# speedup vs baseline: 1.5893x; 1.5893x over previous
import functools

import jax
import jax.numpy as jnp
from jax import lax
from jax.experimental import pallas as pl
from jax.experimental.pallas import tpu as pltpu
from jax.experimental.pallas import tpu_sc as plsc

L = 4096
NUM_WORKERS = 32
ROWS_PER_WORKER = L // NUM_WORKERS
ROWS_PER_CHUNK = 1
NUM_CHUNKS = ROWS_PER_WORKER // ROWS_PER_CHUNK
VECS_PER_ROW = L // 16
NBUF = 8
ROUNDS = NUM_CHUNKS // NBUF


def _sc_lookup(table16, idx):
    mesh = plsc.VectorSubcoreMesh(core_axis_name="c", subcore_axis_name="s")

    scratch = [pltpu.VMEM((16,), jnp.float32)]
    scratch += [pltpu.VMEM((ROWS_PER_CHUNK, L), jnp.int32)] * NBUF
    scratch += [pltpu.VMEM((ROWS_PER_CHUNK, L), jnp.float32)] * NBUF
    scratch += [pltpu.SemaphoreType.DMA] * (2 * NBUF)

    @functools.partial(
        pl.kernel,
        mesh=mesh,
        out_type=jax.ShapeDtypeStruct((L, L), jnp.float32),
        compiler_params=pltpu.CompilerParams(needs_layout_passes=False),
        scratch_types=scratch,
    )
    def k(table_hbm, idx_hbm, out_hbm, tab_v, *bufs):
        idx_b = bufs[:NBUF]
        out_b = bufs[NBUF:2 * NBUF]
        sin = bufs[2 * NBUF:3 * NBUF]
        sout = bufs[3 * NBUF:4 * NBUF]
        wid = lax.axis_index("s") * 2 + lax.axis_index("c")
        row_base = wid * ROWS_PER_WORKER
        pltpu.sync_copy(table_hbm, tab_v)
        tab_vec = tab_v[...]

        for b in range(NBUF):
            pltpu.async_copy(
                idx_hbm.at[pl.ds(row_base + b * ROWS_PER_CHUNK,
                                 ROWS_PER_CHUNK)],
                idx_b[b], sin[b])

        def round_body(r, carry):
            for b in range(NBUF):
                ci = r * NBUF + b
                r0 = row_base + ci * ROWS_PER_CHUNK
                pltpu.make_async_copy(
                    idx_hbm.at[pl.ds(r0, ROWS_PER_CHUNK)],
                    idx_b[b], sin[b]).wait()

                @pl.when(r > 0)
                def _wait_out():
                    pltpu.make_async_copy(
                        out_b[b], out_hbm.at[pl.ds(r0, ROWS_PER_CHUNK)],
                        sout[b]).wait()

                for row in range(ROWS_PER_CHUNK):
                    @plsc.parallel_loop(0, VECS_PER_ROW, 1, unroll=8)
                    def _vec(vi):
                        s = pl.ds(vi * 16, 16)
                        iv = idx_b[b][row, s]
                        out_b[b][row, s] = tab_vec.at[iv].get(
                            mode="promise_in_bounds")

                pltpu.async_copy(
                    out_b[b], out_hbm.at[pl.ds(r0, ROWS_PER_CHUNK)], sout[b])

                @pl.when(ci + NBUF < NUM_CHUNKS)
                def _prefetch():
                    r2 = row_base + (ci + NBUF) * ROWS_PER_CHUNK
                    pltpu.async_copy(
                        idx_hbm.at[pl.ds(r2, ROWS_PER_CHUNK)],
                        idx_b[b], sin[b])

            return carry

        lax.fori_loop(0, ROUNDS, round_body, 0)

        for b in range(NBUF):
            pltpu.make_async_copy(
                out_b[b], out_hbm.at[pl.ds(row_base, ROWS_PER_CHUNK)],
                sout[b]).wait()

    return k(table16, idx)


def kernel(selected_ids, crf_transitions_model):
    idx = selected_ids.astype(jnp.int32)
    flat = crf_transitions_model.reshape(-1)
    table16 = jnp.concatenate([flat, jnp.zeros((1,), jnp.float32)])
    return _sc_lookup(table16, idx)
